# Initial kernel scaffold; baseline (speedup 1.0000x reference)
import jax
import jax.numpy as jnp
from jax.experimental import pallas as pl


def kernel(x, W):
    raise NotImplementedError("write your pallas kernel here")


# speedup vs baseline: 1.3993x; 1.3993x over previous
import functools
import math

import jax
import jax.numpy as jnp
from jax import lax
from jax.experimental import pallas as pl
from jax.experimental.pallas import tpu as pltpu
from jax.experimental.pallas import tpu_sc as plsc

_D = 32
_L = 16
_NC = 2
_NS = 16
_NW = _NC * _NS

_IROW = 128
_GPC = 8
_CHUNK = _IROW * _GPC


def _emb_kernel(n_total):
    n_per_w = n_total // _NW
    n_chunks = n_per_w // _CHUNK
    scale = math.sqrt(float(_D))
    mesh = plsc.VectorSubcoreMesh(core_axis_name="c", subcore_axis_name="s")

    @functools.partial(
        pl.kernel,
        mesh=mesh,
        out_type=jax.ShapeDtypeStruct((n_total, _D), jnp.float32),
        scratch_types=[
            pltpu.VMEM((_GPC, _IROW), jnp.int32),
            pltpu.VMEM((_CHUNK, _D), jnp.float32),
            pltpu.SemaphoreType.DMA,
        ],
        compiler_params=pltpu.CompilerParams(use_tc_tiling_on_sc=False),
    )
    def k(idx_hbm, tbl_hbm, out_hbm, idx_v, rows_v, sem):
        wid = lax.axis_index("s") * _NC + lax.axis_index("c")
        base = wid * n_per_w

        def chunk_body(ci, _):
            off = pl.multiple_of(base + ci * _CHUNK, _CHUNK)
            pltpu.sync_copy(
                idx_hbm.at[pl.ds(pl.multiple_of(off // _IROW, _GPC), _GPC)],
                idx_v,
            )
            copies = [
                pltpu.async_copy(
                    tbl_hbm.at[idx_v.at[g]],
                    rows_v.at[pl.ds(g * _IROW, _IROW)],
                    sem,
                )
                for g in range(_GPC)
            ]
            for c in copies:
                c.wait()

            def scale_body(i, _):
                rows_v[i, pl.ds(0, _L)] = rows_v[i, pl.ds(0, _L)] * scale
                rows_v[i, pl.ds(_L, _L)] = rows_v[i, pl.ds(_L, _L)] * scale
                return 0

            lax.fori_loop(0, _CHUNK, scale_body, 0, unroll=4)
            pltpu.sync_copy(rows_v, out_hbm.at[pl.ds(off, _CHUNK)])
            return 0

        lax.fori_loop(0, n_chunks, chunk_body, 0)

    return k


def kernel(x, W):
    b, h = x.shape
    n_total = b * h
    idx2d = x.reshape(n_total // _IROW, _IROW)
    out = _emb_kernel(n_total)(idx2d, W)
    return out.reshape(b, h, _D)


# speedup vs baseline: 1.4666x; 1.0480x over previous
import functools
import math

import jax
import jax.numpy as jnp
from jax import lax
from jax.experimental import pallas as pl
from jax.experimental.pallas import tpu as pltpu
from jax.experimental.pallas import tpu_sc as plsc

_D = 32
_L = 16
_NC = 2
_NS = 16
_NW = _NC * _NS

_IROW = 128
_GPC = 8
_CHUNK = _IROW * _GPC
_NBUF = 3


def _emb_kernel(n_total):
    n_per_w = n_total // _NW
    n_chunks = n_per_w // _CHUNK
    n_groups = (n_chunks + _NBUF - 1) // _NBUF
    scale = math.sqrt(float(_D))
    mesh = plsc.VectorSubcoreMesh(core_axis_name="c", subcore_axis_name="s")

    @functools.partial(
        pl.kernel,
        mesh=mesh,
        out_type=jax.ShapeDtypeStruct((n_total, _D), jnp.float32),
        scratch_types=[
            pltpu.VMEM((_NBUF, _GPC, _IROW), jnp.int32),
            pltpu.VMEM((_NBUF, _CHUNK, _D), jnp.float32),
            pltpu.SemaphoreType.DMA((_NBUF,)),
            pltpu.SemaphoreType.DMA((_NBUF,)),
        ],
        compiler_params=pltpu.CompilerParams(use_tc_tiling_on_sc=False),
    )
    def k(idx_hbm, tbl_hbm, out_hbm, idx_v, rows_v, gsem, osem):
        wid = lax.axis_index("s") * _NC + lax.axis_index("c")
        base = wid * n_per_w

        def gathers(b):
            return [
                pltpu.make_async_copy(
                    tbl_hbm.at[idx_v.at[b].at[g]],
                    rows_v.at[b].at[pl.ds(g * _IROW, _IROW)],
                    gsem.at[b],
                )
                for g in range(_GPC)
            ]

        def out_copy(ci, b):
            return pltpu.make_async_copy(
                rows_v.at[b],
                out_hbm.at[pl.ds(base + ci * _CHUNK, _CHUNK)],
                osem.at[b],
            )

        def issue(ci, b):
            row = (base + ci * _CHUNK) // _IROW
            pltpu.sync_copy(
                idx_hbm.at[pl.ds(pl.multiple_of(row, _GPC), _GPC)], idx_v.at[b]
            )
            for c in gathers(b):
                c.start()

        for b in range(_NBUF - 1):
            issue(b, b)

        def step(ci, b):
            bp = (b + _NBUF - 1) % _NBUF

            @pl.when(ci >= 1)
            def _():
                out_copy(ci - 1, bp).wait()

            @pl.when(ci + _NBUF - 1 < n_chunks)
            def _():
                issue(ci + _NBUF - 1, bp)

            for c in gathers(b):
                c.wait()

            def scale_body(i, _):
                rows_v[b, i, pl.ds(0, _L)] = rows_v[b, i, pl.ds(0, _L)] * scale
                rows_v[b, i, pl.ds(_L, _L)] = rows_v[b, i, pl.ds(_L, _L)] * scale
                return 0

            lax.fori_loop(0, _CHUNK, scale_body, 0, unroll=8)
            out_copy(ci, b).start()

        def group(cj, _):
            for b in range(_NBUF):
                ci = cj * _NBUF + b

                @pl.when(ci < n_chunks)
                def _():
                    step(ci, b)

            return 0

        lax.fori_loop(0, n_groups, group, 0)
        out_copy(n_chunks - 1, (n_chunks - 1) % _NBUF).wait()

    return k


def kernel(x, W):
    b, h = x.shape
    n_total = b * h
    idx2d = x.reshape(n_total // _IROW, _IROW)
    out = _emb_kernel(n_total)(idx2d, W)
    return out.reshape(b, h, _D)
